# baseline (speedup 1.0000x reference)
import jax
import jax.numpy as jnp
from jax import lax
from jax.experimental import pallas as pl
from jax.experimental.pallas import tpu as pltpu
from jax.experimental.pallas import tpu_sc as plsc

B = 16384
D = 32
NC = 2
NS = 16
NW = NC * NS
CHUNK = B // NW
L = 16
CK = 1024
N_BIG = 1_000_000
N_SMALL = 100_000
NPAD_BIG = 1_000_064
NPAD_SMALL = 100_096
PIE_BIG = 31232
PIE_SMALL = 3072
RING = 256
STG = 4096


def _extract_kernel(user, item, uattr, iattr, utT, itT, uatT, iatT,
                    gu, gi, gua, gia,
                    stg, hcol, hpos, ring, ck0, ck1, dump,
                    semf, semw):
  cid = lax.axis_index("c")
  sid = lax.axis_index("s")
  wid = sid * NC + cid
  is_last = wid == NW - 1

  tabs = (utT, itT, uatT, iatT)
  gbufs = (gu, gi, gua, gia)
  idxs = (user, item, uattr, iattr)
  sizes = (N_BIG, N_BIG, N_SMALL, N_SMALL)
  npads = (NPAD_BIG, NPAD_BIG, NPAD_SMALL, NPAD_SMALL)
  lane = lax.iota(jnp.int32, L)
  rows0 = lane
  rows1 = lane + L
  cks = (ck0, ck1)

  def _wb_wait():
    pltpu.make_async_copy(gu.at[pl.ds(0, D)], dump, semw).wait()

  htot = jnp.int32(0)

  for t in range(4):
    n = sizes[t]
    npad = npads[t]
    pie = PIE_BIG if t < 2 else PIE_SMALL
    tail = n - NW * pie
    lo = pl.multiple_of(wid * pie, 128)
    hi = lax.select(is_last, jnp.int32(n), lo + pie)
    gb = gbufs[t]
    tab = tabs[t]

    np_ = -(-(pie + tail) // CK)
    np_pad = np_ + (np_ % 2)

    def _start(pi, tab=tab, npad=npad, lo=lo):
      s = lax.min(lo + pi * CK, jnp.int32(npad - CK))
      return pl.multiple_of(s, 128)

    def _fetch(pi, buf, tab=tab):
      pltpu.async_copy(tab.at[:, pl.ds(_start(pi), CK)], buf, semf)

    def _fwait(pi, buf, tab=tab):
      pltpu.make_async_copy(tab.at[:, pl.ds(_start(pi), CK)], buf, semf).wait()

    nh = jnp.int32(0)
    for s in range(B // STG):
      pltpu.sync_copy(idxs[t].at[pl.ds(s * STG, STG)], stg)

      def _scan(v, nh_, s=s):
        vec = stg[pl.ds(v * L, L)]
        m = (vec >= lo) & (vec < hi)
        posv = jnp.broadcast_to(s * STG, (L,)).astype(jnp.int32) + v * L + lane
        plsc.store_compressed(hcol.at[pl.ds(nh_, L)], vec - lo, mask=m)
        plsc.store_compressed(hpos.at[pl.ds(nh_, L)], posv, mask=m)
        return nh_ + plsc.all_reduce_population_count(m)[0]

      nh = lax.fori_loop(0, STG // L, _scan, nh)

    ngv = lax.div(nh + (L - 1), jnp.int32(L))

    def _process(pi, buf, h):
      poff = pi * CK
      shift = lo + poff - _start(pi)

      def _vloop(v, h_):
        cvec = hcol[pl.ds(v * L, L)]
        pvec = hpos[pl.ds(v * L, L)]
        valid = (lane + v * L) < nh
        m = valid & (cvec >= poff) & (cvec < poff + CK)
        lcol = cvec - poff + shift
        plsc.store_compressed(stg.at[pl.ds(0, L)], lcol, mask=m)
        plsc.store_compressed(stg.at[pl.ds(L, L)], pvec, mask=m)
        k = plsc.all_reduce_population_count(m)[0]
        lc2 = stg[pl.ds(0, L)]
        pp2 = stg[pl.ds(L, L)]
        for ln in range(L):
          @pl.when(k > ln)
          def _():
            c = jnp.broadcast_to(lc2[ln], (L,))
            p = pp2[ln]
            v0 = plsc.load_gather(buf, [rows0, c])
            v1 = plsc.load_gather(buf, [rows1, c])
            hh = h_ + ln
            slot = (hh & (RING - 1)) * D
            @pl.when(hh >= RING)
            def _w():
              _wb_wait()
            ring[pl.ds(slot, L)] = v0
            ring[pl.ds(slot + L, L)] = v1
            pltpu.async_copy(
                ring.at[pl.ds(slot, D)], gb.at[pl.ds(p * D, D)], semw)
        return h_ + k

      return lax.fori_loop(0, ngv, _vloop, h)

    _fetch(jnp.int32(0), ck0)
    if np_pad > 1:
      _fetch(jnp.int32(1), ck1)

    def _pair(i, h):
      pi0 = 2 * i
      pi1 = 2 * i + 1
      _fwait(pi0, ck0)
      h = _process(pi0, ck0, h)

      @pl.when(pi0 + 2 < np_pad)
      def _():
        _fetch(pi0 + 2, ck0)

      _fwait(pi1, ck1)
      h = _process(pi1, ck1, h)

      @pl.when(pi1 + 2 < np_pad)
      def _():
        _fetch(pi1 + 2, ck1)

      return h

    htot = lax.fori_loop(0, np_pad // 2, _pair, htot)

  def _fin(i, c):
    @pl.when(i < htot)
    def _():
      _wb_wait()
    return c

  lax.fori_loop(0, jnp.int32(RING), _fin, jnp.int32(0))


def _reduce_kernel(gu, gi, gua, gia, out_hbm,
                   bu, bi, bua, bia, out_v, sem):
  wid = lax.axis_index("s") * NC + lax.axis_index("c")
  base = wid * CHUNK

  cps = []
  for src, dst in zip((gu, gi, gua, gia), (bu, bi, bua, bia)):
    cps.append(pltpu.async_copy(src.at[pl.ds(base * D, CHUNK * D)], dst, sem))
  for c in cps:
    c.wait()

  lane = lax.iota(jnp.int32, L)
  lane0 = lane == 0

  @plsc.parallel_loop(0, CHUNK, unroll=8)
  def _row(r):
    o = r * D
    u0 = bu[pl.ds(o, L)]
    u1 = bu[pl.ds(o + L, L)]
    i0 = bi[pl.ds(o, L)]
    i1 = bi[pl.ds(o + L, L)]
    a0 = bua[pl.ds(o, L)]
    a1 = bua[pl.ds(o + L, L)]
    b0 = bia[pl.ds(o, L)]
    b1 = bia[pl.ds(o + L, L)]
    v = (u0 * (i0 + a0) + i0 * b0) + (u1 * (i1 + a1) + i1 * b1)
    s = jnp.broadcast_to(jnp.sum(v), (L,))
    ridx = jnp.broadcast_to(r, (L,)).astype(jnp.int32)
    plsc.store_scatter(out_v, [ridx], s, mask=lane0)

  pltpu.sync_copy(out_v, out_hbm.at[pl.ds(base, CHUNK)])


@jax.jit
def kernel(user, item, user_attributes, item_attributes,
           user_table, item_table, user_attr_table, item_attr_table):
  mesh = plsc.VectorSubcoreMesh(core_axis_name="c", subcore_axis_name="s")
  params = pltpu.CompilerParams(
      needs_layout_passes=False, disable_bounds_checks=True)

  f1 = pl.kernel(
      _extract_kernel,
      out_type=tuple(jax.ShapeDtypeStruct((B * D,), jnp.float32)
                     for _ in range(4)),
      mesh=mesh,
      compiler_params=params,
      scratch_types=[
          pltpu.VMEM((STG,), jnp.int32),
          pltpu.VMEM((B + L,), jnp.int32),
          pltpu.VMEM((B + L,), jnp.int32),
          pltpu.VMEM((RING * D,), jnp.float32),
          pltpu.VMEM((D, CK), jnp.float32),
          pltpu.VMEM((D, CK), jnp.float32),
          pltpu.VMEM((D,), jnp.float32),
          pltpu.SemaphoreType.DMA,
          pltpu.SemaphoreType.DMA,
      ],
  )
  gu, gi, gua, gia = f1(user, item, user_attributes, item_attributes,
                        user_table.T, item_table.T,
                        user_attr_table.T, item_attr_table.T)

  f2 = pl.kernel(
      _reduce_kernel,
      out_type=jax.ShapeDtypeStruct((B,), jnp.float32),
      mesh=mesh,
      compiler_params=params,
      scratch_types=[
          pltpu.VMEM((CHUNK * D,), jnp.float32),
          pltpu.VMEM((CHUNK * D,), jnp.float32),
          pltpu.VMEM((CHUNK * D,), jnp.float32),
          pltpu.VMEM((CHUNK * D,), jnp.float32),
          pltpu.VMEM((CHUNK,), jnp.float32),
          pltpu.SemaphoreType.DMA,
      ],
  )
  return f2(gu, gi, gua, gia)


# speedup vs baseline: 2.7029x; 2.7029x over previous
import jax
import jax.numpy as jnp
from jax import lax
from jax.experimental import pallas as pl
from jax.experimental.pallas import tpu as pltpu
from jax.experimental.pallas import tpu_sc as plsc

B = 16384
D = 32
NC = 2
NS = 16
NW = NC * NS
CHUNK = B // NW
L = 16
CK = 1536
N_BIG = 1_000_000
N_SMALL = 100_000
NPAD_BIG = 1_000_064
NPAD_SMALL = 100_096
PIE_BIG = 31232
PIE_SMALL = 3072
RING = 256
STG = 4096
HCAP = 4096


def _extract_kernel(user, item, uattr, iattr, utT, itT, uatT, iatT,
                    gu, gi, gua, gia,
                    stg, hcol, hpos, hcol2, hpos2, pstart, ring, ck0, ck1, dump,
                    semf, semw):
  cid = lax.axis_index("c")
  sid = lax.axis_index("s")
  wid = sid * NC + cid
  is_last = wid == NW - 1

  tabs = (utT, itT, uatT, iatT)
  gbufs = (gu, gi, gua, gia)
  idxs = (user, item, uattr, iattr)
  sizes = (N_BIG, N_BIG, N_SMALL, N_SMALL)
  npads = (NPAD_BIG, NPAD_BIG, NPAD_SMALL, NPAD_SMALL)
  lane = lax.iota(jnp.int32, L)
  rows0 = lane
  rows1 = lane + L
  cks = (ck0, ck1)

  def _wb_wait():
    pltpu.make_async_copy(gu.at[pl.ds(0, D)], dump, semw).wait()

  htot = jnp.int32(0)

  for t in range(4):
    n = sizes[t]
    npad = npads[t]
    pie = PIE_BIG if t < 2 else PIE_SMALL
    tail = n - NW * pie
    lo = pl.multiple_of(wid * pie, 128)
    hi = lax.select(is_last, jnp.int32(n), lo + pie)
    gb = gbufs[t]
    tab = tabs[t]

    np_ = -(-(pie + tail) // CK)
    np_pad = np_ + (np_ % 2)

    def _start(pi, tab=tab, npad=npad, lo=lo):
      s = lax.min(lo + pi * CK, jnp.int32(npad - CK))
      return pl.multiple_of(s, 128)

    def _fetch(pi, buf, tab=tab):
      pltpu.async_copy(tab.at[:, pl.ds(_start(pi), CK)], buf, semf)

    def _fwait(pi, buf, tab=tab):
      pltpu.make_async_copy(tab.at[:, pl.ds(_start(pi), CK)], buf, semf).wait()

    nh = jnp.int32(0)
    for s in range(B // STG):
      pltpu.sync_copy(idxs[t].at[pl.ds(s * STG, STG)], stg)

      def _scan(v, nh_, s=s):
        vec = stg[pl.ds(v * L, L)]
        m = (vec >= lo) & (vec < hi)
        posv = jnp.broadcast_to(s * STG, (L,)).astype(jnp.int32) + v * L + lane
        plsc.store_compressed(hcol.at[pl.ds(nh_, L)], vec - lo, mask=m)
        plsc.store_compressed(hpos.at[pl.ds(nh_, L)], posv, mask=m)
        return nh_ + plsc.all_reduce_population_count(m)[0]

      nh = lax.fori_loop(0, STG // L, _scan, nh)

    ngv = lax.div(nh + (L - 1), jnp.int32(L))
    no2 = jnp.int32(0)
    for p in range(np_):
      plsc.store_scatter(pstart, [jnp.broadcast_to(jnp.int32(p), (L,))],
                         jnp.broadcast_to(no2, (L,)), mask=lane == 0)

      def _bkt(v, o, p=p):
        cvec = hcol[pl.ds(v * L, L)]
        valid = (lane + v * L) < nh
        m = valid & (cvec >= p * CK) & (cvec < (p + 1) * CK)
        pvec = hpos[pl.ds(v * L, L)]
        plsc.store_compressed(hcol2.at[pl.ds(o, L)], cvec, mask=m)
        plsc.store_compressed(hpos2.at[pl.ds(o, L)], pvec, mask=m)
        return o + plsc.all_reduce_population_count(m)[0]

      no2 = lax.fori_loop(0, ngv, _bkt, no2)
    for p in (np_, np_pad):
      plsc.store_scatter(pstart, [jnp.broadcast_to(jnp.int32(p), (L,))],
                         jnp.broadcast_to(nh, (L,)), mask=lane == 0)

    def _process(pi, buf, h):
      poff = pi * CK
      shift = lo + poff - _start(pi)
      c0 = plsc.load_gather(pstart, [jnp.broadcast_to(pi, (L,))])[0]
      c1 = plsc.load_gather(pstart, [jnp.broadcast_to(pi + 1, (L,))])[0]
      cnt = c1 - c0
      ng2 = lax.div(cnt + (L - 1), jnp.int32(L))

      def _vloop(g, h_):
        base = c0 + g * L
        cvec = hcol2[pl.ds(base, L)]
        pvec = hpos2[pl.ds(base, L)]
        k = lax.min(jnp.int32(L), cnt - g * L)
        lcol = cvec - poff + shift
        for ln in range(L):
          @pl.when(k > ln)
          def _():
            c = jnp.broadcast_to(lcol[ln], (L,))
            p = pvec[ln]
            v0 = plsc.load_gather(buf, [rows0, c])
            v1 = plsc.load_gather(buf, [rows1, c])
            hh = h_ + ln
            slot = (hh & (RING - 1)) * D
            @pl.when(hh >= RING)
            def _w():
              _wb_wait()
            ring[pl.ds(slot, L)] = v0
            ring[pl.ds(slot + L, L)] = v1
            pltpu.async_copy(
                ring.at[pl.ds(slot, D)], gb.at[pl.ds(p * D, D)], semw)
        return h_ + k

      return lax.fori_loop(0, ng2, _vloop, h)

    _fetch(jnp.int32(0), ck0)
    if np_pad > 1:
      _fetch(jnp.int32(1), ck1)

    def _pair(i, h):
      pi0 = 2 * i
      pi1 = 2 * i + 1
      _fwait(pi0, ck0)
      h = _process(pi0, ck0, h)

      @pl.when(pi0 + 2 < np_pad)
      def _():
        _fetch(pi0 + 2, ck0)

      _fwait(pi1, ck1)
      h = _process(pi1, ck1, h)

      @pl.when(pi1 + 2 < np_pad)
      def _():
        _fetch(pi1 + 2, ck1)

      return h

    htot = lax.fori_loop(0, np_pad // 2, _pair, htot)

  def _fin(i, c):
    @pl.when(i < htot)
    def _():
      _wb_wait()
    return c

  lax.fori_loop(0, jnp.int32(RING), _fin, jnp.int32(0))


def _reduce_kernel(gu, gi, gua, gia, out_hbm,
                   bu, bi, bua, bia, out_v, sem):
  wid = lax.axis_index("s") * NC + lax.axis_index("c")
  base = wid * CHUNK

  cps = []
  for src, dst in zip((gu, gi, gua, gia), (bu, bi, bua, bia)):
    cps.append(pltpu.async_copy(src.at[pl.ds(base * D, CHUNK * D)], dst, sem))
  for c in cps:
    c.wait()

  lane = lax.iota(jnp.int32, L)
  lane0 = lane == 0

  @plsc.parallel_loop(0, CHUNK, unroll=8)
  def _row(r):
    o = r * D
    u0 = bu[pl.ds(o, L)]
    u1 = bu[pl.ds(o + L, L)]
    i0 = bi[pl.ds(o, L)]
    i1 = bi[pl.ds(o + L, L)]
    a0 = bua[pl.ds(o, L)]
    a1 = bua[pl.ds(o + L, L)]
    b0 = bia[pl.ds(o, L)]
    b1 = bia[pl.ds(o + L, L)]
    v = (u0 * (i0 + a0) + i0 * b0) + (u1 * (i1 + a1) + i1 * b1)
    s = jnp.broadcast_to(jnp.sum(v), (L,))
    ridx = jnp.broadcast_to(r, (L,)).astype(jnp.int32)
    plsc.store_scatter(out_v, [ridx], s, mask=lane0)

  pltpu.sync_copy(out_v, out_hbm.at[pl.ds(base, CHUNK)])


@jax.jit
def kernel(user, item, user_attributes, item_attributes,
           user_table, item_table, user_attr_table, item_attr_table):
  mesh = plsc.VectorSubcoreMesh(core_axis_name="c", subcore_axis_name="s")
  params = pltpu.CompilerParams(
      needs_layout_passes=False, disable_bounds_checks=True)

  f1 = pl.kernel(
      _extract_kernel,
      out_type=tuple(jax.ShapeDtypeStruct((B * D,), jnp.float32)
                     for _ in range(4)),
      mesh=mesh,
      compiler_params=params,
      scratch_types=[
          pltpu.VMEM((STG,), jnp.int32),
          pltpu.VMEM((HCAP + L,), jnp.int32),
          pltpu.VMEM((HCAP + L,), jnp.int32),
          pltpu.VMEM((HCAP + L,), jnp.int32),
          pltpu.VMEM((HCAP + L,), jnp.int32),
          pltpu.VMEM((64,), jnp.int32),
          pltpu.VMEM((RING * D,), jnp.float32),
          pltpu.VMEM((D, CK), jnp.float32),
          pltpu.VMEM((D, CK), jnp.float32),
          pltpu.VMEM((D,), jnp.float32),
          pltpu.SemaphoreType.DMA,
          pltpu.SemaphoreType.DMA,
      ],
  )
  gu, gi, gua, gia = f1(user, item, user_attributes, item_attributes,
                        user_table.T, item_table.T,
                        user_attr_table.T, item_attr_table.T)

  f2 = pl.kernel(
      _reduce_kernel,
      out_type=jax.ShapeDtypeStruct((B,), jnp.float32),
      mesh=mesh,
      compiler_params=params,
      scratch_types=[
          pltpu.VMEM((CHUNK * D,), jnp.float32),
          pltpu.VMEM((CHUNK * D,), jnp.float32),
          pltpu.VMEM((CHUNK * D,), jnp.float32),
          pltpu.VMEM((CHUNK * D,), jnp.float32),
          pltpu.VMEM((CHUNK,), jnp.float32),
          pltpu.SemaphoreType.DMA,
      ],
  )
  return f2(gu, gi, gua, gia)


# speedup vs baseline: 2.7929x; 1.0333x over previous
import jax
import jax.numpy as jnp
from jax import lax
from jax.experimental import pallas as pl
from jax.experimental.pallas import tpu as pltpu
from jax.experimental.pallas import tpu_sc as plsc

B = 16384
D = 32
NC = 2
NS = 16
NW = NC * NS
CHUNK = B // NW
L = 16
CK = 1536
N_BIG = 1_000_000
N_SMALL = 100_000
NPAD_BIG = 1_000_064
NPAD_SMALL = 100_096
PIE_BIG = 31232
PIE_SMALL = 3072
RING = 256
STG = 4096
HCAP = 4096


def _extract_kernel(user, item, uattr, iattr, utT, itT, uatT, iatT,
                    gu, gi, gua, gia,
                    stg, hcol, hpos, hcol2, hpos2, pstart, ring, ck0, ck1, dump,
                    semf, semw):
  cid = lax.axis_index("c")
  sid = lax.axis_index("s")
  wid = sid * NC + cid
  is_last = wid == NW - 1

  tabs = (utT, itT, uatT, iatT)
  gbufs = (gu, gi, gua, gia)
  idxs = (user, item, uattr, iattr)
  sizes = (N_BIG, N_BIG, N_SMALL, N_SMALL)
  npads = (NPAD_BIG, NPAD_BIG, NPAD_SMALL, NPAD_SMALL)
  lane = lax.iota(jnp.int32, L)
  rows0 = lane
  rows1 = lane + L
  cks = (ck0, ck1)

  def _wb_wait():
    pltpu.make_async_copy(gu.at[pl.ds(0, D)], dump, semw).wait()

  htot = jnp.int32(0)

  for t in range(4):
    n = sizes[t]
    npad = npads[t]
    pie = PIE_BIG if t < 2 else PIE_SMALL
    tail = n - NW * pie
    lo = pl.multiple_of(wid * pie, 128)
    hi = lax.select(is_last, jnp.int32(n), lo + pie)
    gb = gbufs[t]
    tab = tabs[t]

    np_ = -(-(pie + tail) // CK)
    np_pad = np_ + (np_ % 2)

    def _start(pi, tab=tab, npad=npad, lo=lo):
      s = lax.min(lo + pi * CK, jnp.int32(npad - CK))
      return pl.multiple_of(s, 128)

    def _fetch(pi, buf, tab=tab):
      pltpu.async_copy(tab.at[:, pl.ds(_start(pi), CK)], buf, semf)

    def _fwait(pi, buf, tab=tab):
      pltpu.make_async_copy(tab.at[:, pl.ds(_start(pi), CK)], buf, semf).wait()

    HSTG = STG // 2
    nstg = B // HSTG

    def _stage(s_):
      return pltpu.async_copy(
          idxs[t].at[pl.ds(s_ * HSTG, HSTG)],
          stg.at[pl.ds((s_ % 2) * HSTG, HSTG)], semf)

    nh = jnp.int32(0)
    dstg = [_stage(0), _stage(1)]
    for s in range(nstg):
      dstg[s % 2].wait()

      def _scan(v, nh_, s=s):
        vec = stg[pl.ds((s % 2) * HSTG + v * L, L)]
        m = (vec >= lo) & (vec < hi)
        posv = jnp.broadcast_to(s * HSTG, (L,)).astype(jnp.int32) + v * L + lane
        plsc.store_compressed(hcol.at[pl.ds(nh_, L)], vec - lo, mask=m)
        plsc.store_compressed(hpos.at[pl.ds(nh_, L)], posv, mask=m)
        return nh_ + plsc.all_reduce_population_count(m)[0]

      nh = lax.fori_loop(0, HSTG // L, _scan, nh)
      if s + 2 < nstg:
        dstg[s % 2] = _stage(s + 2)

    ngv = lax.div(nh + (L - 1), jnp.int32(L))
    no2 = jnp.int32(0)
    for p in range(np_):
      plsc.store_scatter(pstart, [jnp.broadcast_to(jnp.int32(p), (L,))],
                         jnp.broadcast_to(no2, (L,)), mask=lane == 0)

      def _bkt(v, o, p=p):
        cvec = hcol[pl.ds(v * L, L)]
        valid = (lane + v * L) < nh
        m = valid & (cvec >= p * CK) & (cvec < (p + 1) * CK)
        pvec = hpos[pl.ds(v * L, L)]
        plsc.store_compressed(hcol2.at[pl.ds(o, L)], cvec, mask=m)
        plsc.store_compressed(hpos2.at[pl.ds(o, L)], pvec, mask=m)
        return o + plsc.all_reduce_population_count(m)[0]

      no2 = lax.fori_loop(0, ngv, _bkt, no2)
    for p in (np_, np_pad):
      plsc.store_scatter(pstart, [jnp.broadcast_to(jnp.int32(p), (L,))],
                         jnp.broadcast_to(nh, (L,)), mask=lane == 0)

    def _process(pi, buf, h):
      poff = pi * CK
      shift = lo + poff - _start(pi)
      c0 = plsc.load_gather(pstart, [jnp.broadcast_to(pi, (L,))])[0]
      c1 = plsc.load_gather(pstart, [jnp.broadcast_to(pi + 1, (L,))])[0]
      cnt = c1 - c0
      ng2 = lax.div(cnt + (L - 1), jnp.int32(L))

      def _vloop(g, h_):
        base = c0 + g * L
        cvec = hcol2[pl.ds(base, L)]
        pvec = hpos2[pl.ds(base, L)]
        k = lax.min(jnp.int32(L), cnt - g * L)
        lcol = cvec - poff + shift
        for ln in range(L):
          @pl.when(k > ln)
          def _():
            c = jnp.broadcast_to(lcol[ln], (L,))
            p = pvec[ln]
            v0 = plsc.load_gather(buf, [rows0, c])
            v1 = plsc.load_gather(buf, [rows1, c])
            hh = h_ + ln
            slot = (hh & (RING - 1)) * D
            @pl.when(hh >= RING)
            def _w():
              _wb_wait()
            ring[pl.ds(slot, L)] = v0
            ring[pl.ds(slot + L, L)] = v1
            pltpu.async_copy(
                ring.at[pl.ds(slot, D)], gb.at[pl.ds(p * D, D)], semw)
        return h_ + k

      return lax.fori_loop(0, ng2, _vloop, h)

    _fetch(jnp.int32(0), ck0)
    if np_pad > 1:
      _fetch(jnp.int32(1), ck1)

    def _pair(i, h):
      pi0 = 2 * i
      pi1 = 2 * i + 1
      _fwait(pi0, ck0)
      h = _process(pi0, ck0, h)

      @pl.when(pi0 + 2 < np_pad)
      def _():
        _fetch(pi0 + 2, ck0)

      _fwait(pi1, ck1)
      h = _process(pi1, ck1, h)

      @pl.when(pi1 + 2 < np_pad)
      def _():
        _fetch(pi1 + 2, ck1)

      return h

    htot = lax.fori_loop(0, np_pad // 2, _pair, htot)

  def _fin(i, c):
    @pl.when(i < htot)
    def _():
      _wb_wait()
    return c

  lax.fori_loop(0, jnp.int32(RING), _fin, jnp.int32(0))


def _reduce_kernel(gu, gi, gua, gia, out_hbm,
                   bu, bi, bua, bia, out_v, sem):
  wid = lax.axis_index("s") * NC + lax.axis_index("c")
  base = wid * CHUNK

  cps = []
  for src, dst in zip((gu, gi, gua, gia), (bu, bi, bua, bia)):
    cps.append(pltpu.async_copy(src.at[pl.ds(base * D, CHUNK * D)], dst, sem))
  for c in cps:
    c.wait()

  lane = lax.iota(jnp.int32, L)
  lane0 = lane == 0

  @plsc.parallel_loop(0, CHUNK, unroll=8)
  def _row(r):
    o = r * D
    u0 = bu[pl.ds(o, L)]
    u1 = bu[pl.ds(o + L, L)]
    i0 = bi[pl.ds(o, L)]
    i1 = bi[pl.ds(o + L, L)]
    a0 = bua[pl.ds(o, L)]
    a1 = bua[pl.ds(o + L, L)]
    b0 = bia[pl.ds(o, L)]
    b1 = bia[pl.ds(o + L, L)]
    v = (u0 * (i0 + a0) + i0 * b0) + (u1 * (i1 + a1) + i1 * b1)
    s = jnp.broadcast_to(jnp.sum(v), (L,))
    ridx = jnp.broadcast_to(r, (L,)).astype(jnp.int32)
    plsc.store_scatter(out_v, [ridx], s, mask=lane0)

  pltpu.sync_copy(out_v, out_hbm.at[pl.ds(base, CHUNK)])


@jax.jit
def kernel(user, item, user_attributes, item_attributes,
           user_table, item_table, user_attr_table, item_attr_table):
  mesh = plsc.VectorSubcoreMesh(core_axis_name="c", subcore_axis_name="s")
  params = pltpu.CompilerParams(
      needs_layout_passes=False, disable_bounds_checks=True)

  f1 = pl.kernel(
      _extract_kernel,
      out_type=tuple(jax.ShapeDtypeStruct((B * D,), jnp.float32)
                     for _ in range(4)),
      mesh=mesh,
      compiler_params=params,
      scratch_types=[
          pltpu.VMEM((STG,), jnp.int32),
          pltpu.VMEM((HCAP + L,), jnp.int32),
          pltpu.VMEM((HCAP + L,), jnp.int32),
          pltpu.VMEM((HCAP + L,), jnp.int32),
          pltpu.VMEM((HCAP + L,), jnp.int32),
          pltpu.VMEM((64,), jnp.int32),
          pltpu.VMEM((RING * D,), jnp.float32),
          pltpu.VMEM((D, CK), jnp.float32),
          pltpu.VMEM((D, CK), jnp.float32),
          pltpu.VMEM((D,), jnp.float32),
          pltpu.SemaphoreType.DMA,
          pltpu.SemaphoreType.DMA,
      ],
  )
  gu, gi, gua, gia = f1(user, item, user_attributes, item_attributes,
                        user_table.T, item_table.T,
                        user_attr_table.T, item_attr_table.T)

  f2 = pl.kernel(
      _reduce_kernel,
      out_type=jax.ShapeDtypeStruct((B,), jnp.float32),
      mesh=mesh,
      compiler_params=params,
      scratch_types=[
          pltpu.VMEM((CHUNK * D,), jnp.float32),
          pltpu.VMEM((CHUNK * D,), jnp.float32),
          pltpu.VMEM((CHUNK * D,), jnp.float32),
          pltpu.VMEM((CHUNK * D,), jnp.float32),
          pltpu.VMEM((CHUNK,), jnp.float32),
          pltpu.SemaphoreType.DMA,
      ],
  )
  return f2(gu, gi, gua, gia)


# speedup vs baseline: 2.8592x; 1.0237x over previous
import jax
import jax.numpy as jnp
from jax import lax
from jax.experimental import pallas as pl
from jax.experimental.pallas import tpu as pltpu
from jax.experimental.pallas import tpu_sc as plsc

B = 16384
D = 32
NC = 2
NS = 16
NW = NC * NS
CHUNK = B // NW
L = 16
CK = 1536
N_BIG = 1_000_000
N_SMALL = 100_000
NPAD_BIG = 1_000_064
NPAD_SMALL = 100_096
PIE_BIG = 31232
PIE_SMALL = 3072
RING = 256
STG = 4096
HCAP = 4096


def _extract_kernel(user, item, uattr, iattr, utT, itT, uatT, iatT,
                    gu, gi, gua, gia,
                    stg, hcol, hpos, hcol2, hpos2, pstart, ring, ck0, ck1, dump,
                    semf, semw, semi):
  cid = lax.axis_index("c")
  sid = lax.axis_index("s")
  wid = sid * NC + cid
  is_last = wid == NW - 1

  tabs = (utT, itT, uatT, iatT)
  gbufs = (gu, gi, gua, gia)
  idxs = (user, item, uattr, iattr)
  sizes = (N_BIG, N_BIG, N_SMALL, N_SMALL)
  npads = (NPAD_BIG, NPAD_BIG, NPAD_SMALL, NPAD_SMALL)
  lane = lax.iota(jnp.int32, L)
  rows0 = lane
  rows1 = lane + L
  cks = (ck0, ck1)

  def _wb_wait():
    pltpu.make_async_copy(gu.at[pl.ds(0, D)], dump, semw).wait()

  htot = jnp.int32(0)

  for t in range(4):
    n = sizes[t]
    npad = npads[t]
    pie = PIE_BIG if t < 2 else PIE_SMALL
    tail = n - NW * pie
    lo = pl.multiple_of(wid * pie, 128)
    hi = lax.select(is_last, jnp.int32(n), lo + pie)
    gb = gbufs[t]
    tab = tabs[t]

    np_ = -(-(pie + tail) // CK)
    np_pad = np_ + (np_ % 2)

    def _start(pi, tab=tab, npad=npad, lo=lo):
      s = lax.min(lo + pi * CK, jnp.int32(npad - CK))
      return pl.multiple_of(s, 128)

    def _fetch(pi, buf, tab=tab):
      pltpu.async_copy(tab.at[:, pl.ds(_start(pi), CK)], buf, semf)

    def _fwait(pi, buf, tab=tab):
      pltpu.make_async_copy(tab.at[:, pl.ds(_start(pi), CK)], buf, semf).wait()

    _fetch(jnp.int32(0), ck0)
    if True:
      _fetch(jnp.int32(1), ck1)

    HSTG = STG // 2
    nstg = B // HSTG

    def _stage(s_):
      return pltpu.async_copy(
          idxs[t].at[pl.ds(s_ * HSTG, HSTG)],
          stg.at[pl.ds((s_ % 2) * HSTG, HSTG)], semi)

    nh = jnp.int32(0)
    dstg = [_stage(0), _stage(1)]
    for s in range(nstg):
      dstg[s % 2].wait()

      def _scan(v, nh_, s=s):
        vec = stg[pl.ds((s % 2) * HSTG + v * L, L)]
        m = (vec >= lo) & (vec < hi)
        posv = jnp.broadcast_to(s * HSTG, (L,)).astype(jnp.int32) + v * L + lane
        plsc.store_compressed(hcol.at[pl.ds(nh_, L)], vec - lo, mask=m)
        plsc.store_compressed(hpos.at[pl.ds(nh_, L)], posv, mask=m)
        return nh_ + plsc.all_reduce_population_count(m)[0]

      nh = lax.fori_loop(0, HSTG // L, _scan, nh)
      if s + 2 < nstg:
        dstg[s % 2] = _stage(s + 2)

    ngv = lax.div(nh + (L - 1), jnp.int32(L))
    no2 = jnp.int32(0)
    for p in range(np_):
      plsc.store_scatter(pstart, [jnp.broadcast_to(jnp.int32(p), (L,))],
                         jnp.broadcast_to(no2, (L,)), mask=lane == 0)

      def _bkt(v, o, p=p):
        cvec = hcol[pl.ds(v * L, L)]
        valid = (lane + v * L) < nh
        m = valid & (cvec >= p * CK) & (cvec < (p + 1) * CK)
        pvec = hpos[pl.ds(v * L, L)]
        plsc.store_compressed(hcol2.at[pl.ds(o, L)], cvec, mask=m)
        plsc.store_compressed(hpos2.at[pl.ds(o, L)], pvec, mask=m)
        return o + plsc.all_reduce_population_count(m)[0]

      no2 = lax.fori_loop(0, ngv, _bkt, no2)
    for p in (np_, np_pad):
      plsc.store_scatter(pstart, [jnp.broadcast_to(jnp.int32(p), (L,))],
                         jnp.broadcast_to(nh, (L,)), mask=lane == 0)

    def _process(pi, buf, h):
      poff = pi * CK
      shift = lo + poff - _start(pi)
      c0 = plsc.load_gather(pstart, [jnp.broadcast_to(pi, (L,))])[0]
      c1 = plsc.load_gather(pstart, [jnp.broadcast_to(pi + 1, (L,))])[0]
      cnt = c1 - c0
      ng2 = lax.div(cnt + (L - 1), jnp.int32(L))

      def _vloop(g, h_):
        base = c0 + g * L
        cvec = hcol2[pl.ds(base, L)]
        pvec = hpos2[pl.ds(base, L)]
        k = lax.min(jnp.int32(L), cnt - g * L)
        lcol = cvec - poff + shift
        for ln in range(L):
          @pl.when(k > ln)
          def _():
            c = jnp.broadcast_to(lcol[ln], (L,))
            p = pvec[ln]
            v0 = plsc.load_gather(buf, [rows0, c])
            v1 = plsc.load_gather(buf, [rows1, c])
            hh = h_ + ln
            slot = (hh & (RING - 1)) * D
            @pl.when(hh >= RING)
            def _w():
              _wb_wait()
            ring[pl.ds(slot, L)] = v0
            ring[pl.ds(slot + L, L)] = v1
            pltpu.async_copy(
                ring.at[pl.ds(slot, D)], gb.at[pl.ds(p * D, D)], semw)
        return h_ + k

      return lax.fori_loop(0, ng2, _vloop, h)

    def _pair(i, h):
      pi0 = 2 * i
      pi1 = 2 * i + 1
      _fwait(pi0, ck0)
      h = _process(pi0, ck0, h)

      @pl.when(pi0 + 2 < np_pad)
      def _():
        _fetch(pi0 + 2, ck0)

      _fwait(pi1, ck1)
      h = _process(pi1, ck1, h)

      @pl.when(pi1 + 2 < np_pad)
      def _():
        _fetch(pi1 + 2, ck1)

      return h

    htot = lax.fori_loop(0, np_pad // 2, _pair, htot)

  def _fin(i, c):
    @pl.when(i < htot)
    def _():
      _wb_wait()
    return c

  lax.fori_loop(0, jnp.int32(RING), _fin, jnp.int32(0))


def _reduce_kernel(gu, gi, gua, gia, out_hbm,
                   bu, bi, bua, bia, out_v, sem):
  wid = lax.axis_index("s") * NC + lax.axis_index("c")
  base = wid * CHUNK

  cps = []
  for src, dst in zip((gu, gi, gua, gia), (bu, bi, bua, bia)):
    cps.append(pltpu.async_copy(src.at[pl.ds(base * D, CHUNK * D)], dst, sem))
  for c in cps:
    c.wait()

  lane = lax.iota(jnp.int32, L)
  lane0 = lane == 0

  @plsc.parallel_loop(0, CHUNK, unroll=8)
  def _row(r):
    o = r * D
    u0 = bu[pl.ds(o, L)]
    u1 = bu[pl.ds(o + L, L)]
    i0 = bi[pl.ds(o, L)]
    i1 = bi[pl.ds(o + L, L)]
    a0 = bua[pl.ds(o, L)]
    a1 = bua[pl.ds(o + L, L)]
    b0 = bia[pl.ds(o, L)]
    b1 = bia[pl.ds(o + L, L)]
    v = (u0 * (i0 + a0) + i0 * b0) + (u1 * (i1 + a1) + i1 * b1)
    s = jnp.broadcast_to(jnp.sum(v), (L,))
    ridx = jnp.broadcast_to(r, (L,)).astype(jnp.int32)
    plsc.store_scatter(out_v, [ridx], s, mask=lane0)

  pltpu.sync_copy(out_v, out_hbm.at[pl.ds(base, CHUNK)])


@jax.jit
def kernel(user, item, user_attributes, item_attributes,
           user_table, item_table, user_attr_table, item_attr_table):
  mesh = plsc.VectorSubcoreMesh(core_axis_name="c", subcore_axis_name="s")
  params = pltpu.CompilerParams(
      needs_layout_passes=False, disable_bounds_checks=True)

  f1 = pl.kernel(
      _extract_kernel,
      out_type=tuple(jax.ShapeDtypeStruct((B * D,), jnp.float32)
                     for _ in range(4)),
      mesh=mesh,
      compiler_params=params,
      scratch_types=[
          pltpu.VMEM((STG,), jnp.int32),
          pltpu.VMEM((HCAP + L,), jnp.int32),
          pltpu.VMEM((HCAP + L,), jnp.int32),
          pltpu.VMEM((HCAP + L,), jnp.int32),
          pltpu.VMEM((HCAP + L,), jnp.int32),
          pltpu.VMEM((64,), jnp.int32),
          pltpu.VMEM((RING * D,), jnp.float32),
          pltpu.VMEM((D, CK), jnp.float32),
          pltpu.VMEM((D, CK), jnp.float32),
          pltpu.VMEM((D,), jnp.float32),
          pltpu.SemaphoreType.DMA,
          pltpu.SemaphoreType.DMA,
          pltpu.SemaphoreType.DMA,
      ],
  )
  gu, gi, gua, gia = f1(user, item, user_attributes, item_attributes,
                        user_table.T, item_table.T,
                        user_attr_table.T, item_attr_table.T)

  f2 = pl.kernel(
      _reduce_kernel,
      out_type=jax.ShapeDtypeStruct((B,), jnp.float32),
      mesh=mesh,
      compiler_params=params,
      scratch_types=[
          pltpu.VMEM((CHUNK * D,), jnp.float32),
          pltpu.VMEM((CHUNK * D,), jnp.float32),
          pltpu.VMEM((CHUNK * D,), jnp.float32),
          pltpu.VMEM((CHUNK * D,), jnp.float32),
          pltpu.VMEM((CHUNK,), jnp.float32),
          pltpu.SemaphoreType.DMA,
      ],
  )
  return f2(gu, gi, gua, gia)
